# baseline (speedup 1.0000x reference)
import functools

import jax
import jax.numpy as jnp
from jax import lax
from jax.experimental import pallas as pl
from jax.experimental.pallas import tpu as pltpu
from jax.experimental.pallas import tpu_sc as plsc

NC = 2
NS = 16
L = 16
NW = NC * NS
D = 64
CH = 128
BR = 10000


def _repack_body(re_ref, im_ref, out_ref):
    out_ref[:, :D] = re_ref[...]
    out_ref[:, D:] = im_ref[...]


def _repack(ent_re, ent_im):
    n = ent_re.shape[0]
    return pl.pallas_call(
        _repack_body,
        grid=(n // BR,),
        in_specs=[
            pl.BlockSpec((BR, D), lambda i: (i, 0)),
            pl.BlockSpec((BR, D), lambda i: (i, 0)),
        ],
        out_specs=pl.BlockSpec((BR, 2 * D), lambda i: (i, 0)),
        out_shape=jax.ShapeDtypeStruct((n, 2 * D), jnp.float32),
    )(ent_re, ent_im)


def _body(pk_hbm, ent_hbm, rel_hbm, out_hbm,
          idx_v0, idx_v1, eh_v0, eh_v1, et_v0, et_v1, rl_v0, rl_v1,
          out_v, sem0, sem1, *, n_chunks):
    wid = lax.axis_index("s") * NC + lax.axis_index("c")
    rows0 = jnp.arange(L, dtype=jnp.int32)
    idx_bufs = (idx_v0, idx_v1)
    eh_bufs = (eh_v0, eh_v1)
    et_bufs = (et_v0, et_v1)
    rl_bufs = (rl_v0, rl_v1)
    sems = (sem0, sem1)

    def fire(c, b):
        pltpu.sync_copy(pk_hbm.at[wid * n_chunks + c], idx_bufs[b])
        pltpu.async_copy(ent_hbm.at[idx_bufs[b].at[0]], eh_bufs[b], sems[b])
        pltpu.async_copy(ent_hbm.at[idx_bufs[b].at[1]], et_bufs[b], sems[b])
        pltpu.async_copy(rel_hbm.at[idx_bufs[b].at[2]], rl_bufs[b], sems[b])

    def drain(b):
        pltpu.make_async_copy(ent_hbm.at[pl.ds(0, CH)], eh_bufs[b], sems[b]).wait()
        pltpu.make_async_copy(ent_hbm.at[pl.ds(0, CH)], et_bufs[b], sems[b]).wait()
        pltpu.make_async_copy(rel_hbm.at[pl.ds(0, CH)], rl_bufs[b], sems[b]).wait()

    fire(0, 0)
    for c in range(n_chunks):
        b = c % 2
        if c + 1 < n_chunks:
            fire(c + 1, 1 - b)
        drain(b)
        eh_v, et_v, rl_v = eh_bufs[b], et_bufs[b], rl_bufs[b]

        def group_body(g, _):
            rows = g * L + rows0

            def d_body(j, accs):
                acc0, acc1 = accs
                d = j * 2

                def term(cols):
                    cols_im = cols + D
                    ehre = plsc.load_gather(eh_v, [rows, cols])
                    ehim = plsc.load_gather(eh_v, [rows, cols_im])
                    etre = plsc.load_gather(et_v, [rows, cols])
                    etim = plsc.load_gather(et_v, [rows, cols_im])
                    rre = plsc.load_gather(rl_v, [rows, cols])
                    rim = plsc.load_gather(rl_v, [rows, cols_im])
                    return (rre * (ehre * etre + ehim * etim)
                            + rim * (ehre * etim - ehim * etre))

                cols0 = jnp.full((L,), d, dtype=jnp.int32)
                return (acc0 + term(cols0), acc1 + term(cols0 + 1))

            zero = jnp.zeros((L,), jnp.float32)
            acc0, acc1 = lax.fori_loop(0, D // 2, d_body, (zero, zero))
            out_v[pl.ds(g * L, L)] = acc0 + acc1
            return 0

        lax.fori_loop(0, CH // L, group_body, 0)
        pltpu.sync_copy(out_v, out_hbm.at[pl.ds((wid * n_chunks + c) * CH, CH)])


def kernel(hs, rs, ts, ent_re, ent_im, rel_re, rel_im):
    batch = hs.shape[0]
    n_chunks = batch // NW // CH
    ent = _repack(ent_re, ent_im)
    rel = jnp.concatenate([rel_re, rel_im], axis=1)
    pk = jnp.stack([hs, ts, rs], axis=0)
    pk = pk.reshape(3, batch // CH, CH).transpose(1, 0, 2)
    mesh = plsc.VectorSubcoreMesh(core_axis_name="c", subcore_axis_name="s")
    k = pl.kernel(
        functools.partial(_body, n_chunks=n_chunks),
        out_type=jax.ShapeDtypeStruct((batch,), jnp.float32),
        mesh=mesh,
        compiler_params=pltpu.CompilerParams(needs_layout_passes=False),
        scratch_types=[
            pltpu.VMEM((3, CH), jnp.int32),
            pltpu.VMEM((3, CH), jnp.int32),
            pltpu.VMEM((CH, 2 * D), jnp.float32),
            pltpu.VMEM((CH, 2 * D), jnp.float32),
            pltpu.VMEM((CH, 2 * D), jnp.float32),
            pltpu.VMEM((CH, 2 * D), jnp.float32),
            pltpu.VMEM((CH, 2 * D), jnp.float32),
            pltpu.VMEM((CH, 2 * D), jnp.float32),
            pltpu.VMEM((CH,), jnp.float32),
            pltpu.SemaphoreType.DMA,
            pltpu.SemaphoreType.DMA,
        ],
    )
    return k(pk, ent, rel)


# speedup vs baseline: 1.2503x; 1.2503x over previous
import functools

import jax
import jax.numpy as jnp
from jax import lax
from jax.experimental import pallas as pl
from jax.experimental.pallas import tpu as pltpu
from jax.experimental.pallas import tpu_sc as plsc

NC = 2
NS = 16
L = 16
NW = NC * NS
D = 64
CH = 128
BR = 10000


def _repack_body(re_ref, im_ref, out_ref):
    out_ref[:, :D] = re_ref[...]
    out_ref[:, D:] = im_ref[...]


def _repack(ent_re, ent_im):
    n = ent_re.shape[0]
    return pl.pallas_call(
        _repack_body,
        grid=(n // BR,),
        in_specs=[
            pl.BlockSpec((BR, D), lambda i: (i, 0)),
            pl.BlockSpec((BR, D), lambda i: (i, 0)),
        ],
        out_specs=pl.BlockSpec((BR, 2 * D), lambda i: (i, 0)),
        out_shape=jax.ShapeDtypeStruct((n, 2 * D), jnp.float32),
    )(ent_re, ent_im)


def _body(pk_hbm, ent_hbm, rel_hbm, out_hbm,
          idx_v0, idx_v1, eh_v0, eh_v1, et_v0, et_v1, rl_v0, rl_v1,
          out_v, sem0, sem1, *, n_chunks):
    wid = lax.axis_index("s") * NC + lax.axis_index("c")
    rows0 = jnp.arange(L, dtype=jnp.int32)
    idx_bufs = (idx_v0, idx_v1)
    eh_bufs = (eh_v0, eh_v1)
    et_bufs = (et_v0, et_v1)
    rl_bufs = (rl_v0, rl_v1)
    sems = (sem0, sem1)

    def fire(c, b):
        pltpu.sync_copy(pk_hbm.at[wid * n_chunks + c], idx_bufs[b])
        pltpu.async_copy(ent_hbm.at[idx_bufs[b].at[0]], eh_bufs[b], sems[b])
        pltpu.async_copy(ent_hbm.at[idx_bufs[b].at[1]], et_bufs[b], sems[b])
        pltpu.async_copy(rel_hbm.at[idx_bufs[b].at[2]], rl_bufs[b], sems[b])

    def drain(b):
        pltpu.make_async_copy(ent_hbm.at[pl.ds(0, CH)], eh_bufs[b], sems[b]).wait()
        pltpu.make_async_copy(ent_hbm.at[pl.ds(0, CH)], et_bufs[b], sems[b]).wait()
        pltpu.make_async_copy(rel_hbm.at[pl.ds(0, CH)], rl_bufs[b], sems[b]).wait()

    fire(0, 0)
    for c in range(n_chunks):
        b = c % 2
        if c + 1 < n_chunks:
            fire(c + 1, 1 - b)
        drain(b)
        eh_v, et_v, rl_v = eh_bufs[b], et_bufs[b], rl_bufs[b]

        def group_body(g, _):
            rows = g * L + rows0

            def d_body(j, accs):
                acc0, acc1 = accs
                d = j * 2

                def term(cols):
                    cols_im = cols + D
                    ehre = plsc.load_gather(eh_v, [rows, cols])
                    ehim = plsc.load_gather(eh_v, [rows, cols_im])
                    etre = plsc.load_gather(et_v, [rows, cols])
                    etim = plsc.load_gather(et_v, [rows, cols_im])
                    rre = plsc.load_gather(rl_v, [rows, cols])
                    rim = plsc.load_gather(rl_v, [rows, cols_im])
                    return (rre * (ehre * etre + ehim * etim)
                            + rim * (ehre * etim - ehim * etre))

                cols0 = jnp.full((L,), d, dtype=jnp.int32)
                return (acc0 + term(cols0), acc1 + term(cols0 + 1))

            zero = jnp.zeros((L,), jnp.float32)
            acc0, acc1 = lax.fori_loop(0, D // 2, d_body, (zero, zero))
            out_v[pl.ds(g * L, L)] = acc0 + acc1
            return 0

        lax.fori_loop(0, CH // L, group_body, 0)
        pltpu.sync_copy(out_v, out_hbm.at[pl.ds((wid * n_chunks + c) * CH, CH)])


def kernel(hs, rs, ts, ent_re, ent_im, rel_re, rel_im):
    batch = hs.shape[0]
    n_chunks = batch // NW // CH
    ent = jnp.concatenate([ent_re, ent_im], axis=1) + jnp.float32(0.0)
    rel = jnp.concatenate([rel_re, rel_im], axis=1)
    pk = jnp.stack([hs, ts, rs], axis=0)
    pk = pk.reshape(3, batch // CH, CH).transpose(1, 0, 2)
    mesh = plsc.VectorSubcoreMesh(core_axis_name="c", subcore_axis_name="s")
    k = pl.kernel(
        functools.partial(_body, n_chunks=n_chunks),
        out_type=jax.ShapeDtypeStruct((batch,), jnp.float32),
        mesh=mesh,
        compiler_params=pltpu.CompilerParams(needs_layout_passes=False),
        scratch_types=[
            pltpu.VMEM((3, CH), jnp.int32),
            pltpu.VMEM((3, CH), jnp.int32),
            pltpu.VMEM((CH, 2 * D), jnp.float32),
            pltpu.VMEM((CH, 2 * D), jnp.float32),
            pltpu.VMEM((CH, 2 * D), jnp.float32),
            pltpu.VMEM((CH, 2 * D), jnp.float32),
            pltpu.VMEM((CH, 2 * D), jnp.float32),
            pltpu.VMEM((CH, 2 * D), jnp.float32),
            pltpu.VMEM((CH,), jnp.float32),
            pltpu.SemaphoreType.DMA,
            pltpu.SemaphoreType.DMA,
        ],
    )
    return k(pk, ent, rel)


# speedup vs baseline: 2.1929x; 1.7540x over previous
import functools

import jax
import jax.numpy as jnp
from jax import lax
from jax.experimental import pallas as pl
from jax.experimental.pallas import tpu as pltpu
from jax.experimental.pallas import tpu_sc as plsc

NC = 2
NS = 16
L = 16
NW = NC * NS
D = 64
SUB = 8
CH = 64
NIDX = 5


def _body(pk_hbm, ere_hbm, eim_hbm, rel_hbm, out_hbm,
          idx_v0, idx_v1, ehre_v0, ehre_v1, ehim_v0, ehim_v1,
          etre_v0, etre_v1, etim_v0, etim_v1, rl_v0, rl_v1,
          out_v, sem0, sem1, rsem0, rsem1, *, n_chunks):
    wid = lax.axis_index("s") * NC + lax.axis_index("c")
    rows0 = jnp.arange(L, dtype=jnp.int32)
    idx_bufs = (idx_v0, idx_v1)
    ehre_bufs = (ehre_v0, ehre_v1)
    ehim_bufs = (ehim_v0, ehim_v1)
    etre_bufs = (etre_v0, etre_v1)
    etim_bufs = (etim_v0, etim_v1)
    rl_bufs = (rl_v0, rl_v1)
    sems = (sem0, sem1)
    rsems = (rsem0, rsem1)

    def fire(c, b):
        idx_v = idx_bufs[b]
        pltpu.sync_copy(pk_hbm.at[wid * n_chunks + c], idx_v)
        pltpu.async_copy(rel_hbm.at[idx_v.at[4]], rl_bufs[b], rsems[b])
        for g in range(CH // L):
            hb_vec = idx_v[0, pl.ds(g * L, L)]
            tb_vec = idx_v[1, pl.ds(g * L, L)]
            hsub_vec = idx_v[2, pl.ds(g * L, L)]
            tsub_vec = idx_v[3, pl.ds(g * L, L)]
            for i in range(L):
                r = g * L + i
                hb = hb_vec[i]
                tb = tb_vec[i]
                hsb = hsub_vec[i]
                tsb = tsub_vec[i]
                pltpu.async_copy(ere_hbm.at[hb, hsb], ehre_bufs[b].at[r], sems[b])
                pltpu.async_copy(eim_hbm.at[hb, hsb], ehim_bufs[b].at[r], sems[b])
                pltpu.async_copy(ere_hbm.at[tb, tsb], etre_bufs[b].at[r], sems[b])
                pltpu.async_copy(eim_hbm.at[tb, tsb], etim_bufs[b].at[r], sems[b])

    def drain(b):
        pltpu.make_async_copy(ere_hbm.at[0], ehre_bufs[b], sems[b]).wait()
        pltpu.make_async_copy(eim_hbm.at[0], ehim_bufs[b], sems[b]).wait()
        pltpu.make_async_copy(ere_hbm.at[0], etre_bufs[b], sems[b]).wait()
        pltpu.make_async_copy(eim_hbm.at[0], etim_bufs[b], sems[b]).wait()
        pltpu.make_async_copy(rel_hbm.at[pl.ds(0, CH)], rl_bufs[b], rsems[b]).wait()

    def compute(c, b):
        ehre_v, ehim_v = ehre_bufs[b], ehim_bufs[b]
        etre_v, etim_v = etre_bufs[b], etim_bufs[b]
        rl_v = rl_bufs[b]

        def group_body(g, _):
            rows = g * L + rows0

            def d_body(d, acc):
                cols = jnp.full((L,), d, dtype=jnp.int32)
                ehre = plsc.load_gather(ehre_v, [rows, cols])
                ehim = plsc.load_gather(ehim_v, [rows, cols])
                etre = plsc.load_gather(etre_v, [rows, cols])
                etim = plsc.load_gather(etim_v, [rows, cols])
                rre = plsc.load_gather(rl_v, [rows, cols])
                rim = plsc.load_gather(rl_v, [rows, cols + D])
                return (acc + rre * (ehre * etre + ehim * etim)
                        + rim * (ehre * etim - ehim * etre))

            acc = lax.fori_loop(0, D, d_body, jnp.zeros((L,), jnp.float32))
            out_v[pl.ds(g * L, L)] = acc
            return 0

        lax.fori_loop(0, CH // L, group_body, 0)
        pltpu.sync_copy(out_v, out_hbm.at[pl.ds((wid * n_chunks + c) * CH, CH)])

    fire(0, 0)

    def pair_body(c2, _):
        c = c2 * 2
        fire(c + 1, 1)
        drain(0)
        compute(c, 0)

        @pl.when(c2 < n_chunks // 2 - 1)
        def _():
            fire(c + 2, 0)

        drain(1)
        compute(c + 1, 1)
        return 0

    lax.fori_loop(0, n_chunks // 2, pair_body, 0)


def kernel(hs, rs, ts, ent_re, ent_im, rel_re, rel_im):
    batch = hs.shape[0]
    n_chunks = batch // NW // CH
    num_ent = ent_re.shape[0]
    ere3 = ent_re.reshape(num_ent // SUB, SUB, D)
    eim3 = ent_im.reshape(num_ent // SUB, SUB, D)
    rel = jnp.concatenate([rel_re, rel_im], axis=1)
    hb = lax.shift_right_logical(hs, 3)
    hsub = lax.bitwise_and(hs, 7)
    tb = lax.shift_right_logical(ts, 3)
    tsub = lax.bitwise_and(ts, 7)
    pk = jnp.stack([hb, tb, hsub, tsub, rs], axis=0)
    pk = pk.reshape(NIDX, batch // CH, CH).transpose(1, 0, 2)
    mesh = plsc.VectorSubcoreMesh(core_axis_name="c", subcore_axis_name="s")
    k = pl.kernel(
        functools.partial(_body, n_chunks=n_chunks),
        out_type=jax.ShapeDtypeStruct((batch,), jnp.float32),
        mesh=mesh,
        compiler_params=pltpu.CompilerParams(needs_layout_passes=False),
        scratch_types=[
            pltpu.VMEM((NIDX, CH), jnp.int32),
            pltpu.VMEM((NIDX, CH), jnp.int32),
            pltpu.VMEM((CH, D), jnp.float32),
            pltpu.VMEM((CH, D), jnp.float32),
            pltpu.VMEM((CH, D), jnp.float32),
            pltpu.VMEM((CH, D), jnp.float32),
            pltpu.VMEM((CH, D), jnp.float32),
            pltpu.VMEM((CH, D), jnp.float32),
            pltpu.VMEM((CH, D), jnp.float32),
            pltpu.VMEM((CH, D), jnp.float32),
            pltpu.VMEM((CH, 2 * D), jnp.float32),
            pltpu.VMEM((CH, 2 * D), jnp.float32),
            pltpu.VMEM((CH,), jnp.float32),
            pltpu.SemaphoreType.DMA,
            pltpu.SemaphoreType.DMA,
            pltpu.SemaphoreType.DMA,
            pltpu.SemaphoreType.DMA,
        ],
    )
    return k(pk, ere3, eim3, rel)
